# baseline (speedup 1.0000x reference)
import functools
import jax
import jax.numpy as jnp
from jax import lax
from jax.experimental import pallas as pl
from jax.experimental.pallas import tpu as pltpu
from jax.experimental.pallas import tpu_sc as plsc

NE = 8
ALPHA = 1.702
LIMIT = 7.0
BM = 256
NB = 2048 * 2 // BM + NE
NBM = NB * BM
NW = 32


def _router_body(x_ref, wT_ref, b_ref, idx_ref, wts_ref):
    x = x_ref[...]
    logits = jnp.dot(x, wT_ref[...], preferred_element_type=jnp.float32)
    logits = logits + b_ref[...]
    t = logits.shape[0]
    iota = lax.broadcasted_iota(jnp.int32, (t, NE), 1)
    m1 = jnp.max(logits, axis=1, keepdims=True)
    i1 = jnp.min(jnp.where(logits == m1, iota, NE), axis=1, keepdims=True)
    masked = jnp.where(iota == i1, -jnp.inf, logits)
    m2 = jnp.max(masked, axis=1, keepdims=True)
    i2 = jnp.min(jnp.where(masked == m2, iota, NE), axis=1, keepdims=True)
    r = jnp.exp(m2 - m1)
    w1 = 1.0 / (1.0 + r)
    idx_ref[...] = jnp.concatenate([i1, i2], axis=1)
    wts_ref[...] = jnp.concatenate([w1, 1.0 - w1], axis=1)


def _gmm_body(be_ref, xg_ref, wg_ref, wu_ref, wd_ref, bg_ref, bu_ref, bd_ref,
              rww_ref, o_ref):
    x = xg_ref[...]
    gate = jnp.dot(x, wg_ref[0], preferred_element_type=jnp.float32) + bg_ref[0]
    up = jnp.dot(x, wu_ref[0], preferred_element_type=jnp.float32) + bu_ref[0]
    gate = jnp.minimum(gate, LIMIT)
    up = jnp.clip(up, -LIMIT, LIMIT)
    glu = gate * (1.0 / (1.0 + jnp.exp(-ALPHA * gate)))
    act = (up + 1.0) * glu
    y = jnp.dot(act, wd_ref[0], preferred_element_type=jnp.float32) + bd_ref[0]
    o_ref[...] = y * rww_ref[0]


def _sc_gather_body(x_hbm, tok_hbm, out_hbm, idx_v, rows_v, sem):
    wid = lax.axis_index("s") * 2 + lax.axis_index("c")
    n = NBM // NW // 2
    for c in range(2):
        base = wid * (2 * n) + c * n
        pltpu.sync_copy(tok_hbm.at[pl.ds(base, n)], idx_v)
        pltpu.async_copy(x_hbm.at[idx_v], rows_v, sem).wait()
        pltpu.sync_copy(rows_v, out_hbm.at[pl.ds(base, n)])


def _sc_combine_body(yg_hbm, p0_hbm, p1_hbm, out_hbm, i0_v, i1_v, r0_v, r1_v,
                     sem):
    wid = lax.axis_index("s") * 2 + lax.axis_index("c")
    nt = 2048 // NW
    h = 768
    base = wid * nt
    pltpu.sync_copy(p0_hbm.at[pl.ds(base, nt)], i0_v)
    pltpu.sync_copy(p1_hbm.at[pl.ds(base, nt)], i1_v)
    pltpu.async_copy(yg_hbm.at[i0_v], r0_v, sem).wait()
    pltpu.async_copy(yg_hbm.at[i1_v], r1_v, sem).wait()

    def row(i, _):
        def chunk(j, _):
            r0_v[i, pl.ds(j * 16, 16)] = (r0_v[i, pl.ds(j * 16, 16)]
                                          + r1_v[i, pl.ds(j * 16, 16)])
            return _
        return lax.fori_loop(0, h // 16, chunk, _)

    lax.fori_loop(0, nt, row, None)
    pltpu.sync_copy(r0_v, out_hbm.at[pl.ds(base, nt)])


def kernel(hidden_states, router_weight, router_bias, gate_up_proj,
           gate_up_proj_bias, down_proj, down_proj_bias):
    b, s, h = hidden_states.shape
    t = b * s
    i_dim = down_proj.shape[1]
    x2 = hidden_states.reshape(t, h)

    top_idx, top_w = pl.pallas_call(
        _router_body,
        out_shape=(jax.ShapeDtypeStruct((t, 2), jnp.int32),
                   jax.ShapeDtypeStruct((t, 2), jnp.float32)),
    )(x2, router_weight.T, router_bias.reshape(1, NE))

    e_flat = top_idx.reshape(-1)
    w_flat = top_w.reshape(-1)
    tok = lax.broadcasted_iota(jnp.int32, (t, 2), 0).reshape(-1)
    perm = jnp.argsort(e_flat, stable=True)
    e_sorted = e_flat[perm]
    sizes = jnp.sum((e_flat[:, None] == jnp.arange(NE)[None, :]), axis=0)
    starts = jnp.concatenate([jnp.zeros((1,), jnp.int32),
                              jnp.cumsum(sizes)[:-1].astype(jnp.int32)])
    nb_e = (sizes + BM - 1) // BM
    bounds = jnp.cumsum(nb_e)
    blk_start = bounds - nb_e
    block_expert = jnp.minimum(
        jnp.sum(jnp.arange(NB)[:, None] >= bounds[None, :], axis=1),
        NE - 1).astype(jnp.int32)
    rank = jnp.arange(2 * t, dtype=jnp.int32) - starts[e_sorted]
    pos = (blk_start[e_sorted] * BM).astype(jnp.int32) + rank
    row_token = jnp.zeros((NBM,), jnp.int32).at[pos].set(tok[perm])
    row_weight = jnp.zeros((NBM,), jnp.float32).at[pos].set(w_flat[perm])
    posj = jnp.zeros((2 * t,), jnp.int32).at[perm].set(pos).reshape(t, 2)
    p0 = posj[:, 0]
    p1 = posj[:, 1]

    mesh = plsc.VectorSubcoreMesh(core_axis_name="c", subcore_axis_name="s")
    n_g = NBM // NW // 2
    xg = pl.kernel(
        _sc_gather_body,
        out_type=jax.ShapeDtypeStruct((NBM, h), jnp.float32),
        mesh=mesh,
        scratch_types=[pltpu.VMEM((n_g,), jnp.int32),
                       pltpu.VMEM((n_g, h), jnp.float32),
                       pltpu.SemaphoreType.DMA],
    )(x2, row_token)

    gu = gate_up_proj.reshape(NE, h, i_dim, 2)
    wg = gu[..., 0]
    wu = gu[..., 1]
    bgu = gate_up_proj_bias.reshape(NE, 1, i_dim, 2)
    bg = bgu[..., 0]
    bu = bgu[..., 1]
    bd = down_proj_bias.reshape(NE, 1, h)
    rww = row_weight.reshape(NB, BM, 1)

    yg = pl.pallas_call(
        _gmm_body,
        grid_spec=pltpu.PrefetchScalarGridSpec(
            num_scalar_prefetch=1,
            grid=(NB,),
            in_specs=[
                pl.BlockSpec((BM, h), lambda n, be: (n, 0)),
                pl.BlockSpec((1, h, i_dim), lambda n, be: (be[n], 0, 0)),
                pl.BlockSpec((1, h, i_dim), lambda n, be: (be[n], 0, 0)),
                pl.BlockSpec((1, i_dim, h), lambda n, be: (be[n], 0, 0)),
                pl.BlockSpec((1, 1, i_dim), lambda n, be: (be[n], 0, 0)),
                pl.BlockSpec((1, 1, i_dim), lambda n, be: (be[n], 0, 0)),
                pl.BlockSpec((1, 1, h), lambda n, be: (be[n], 0, 0)),
                pl.BlockSpec((1, BM, 1), lambda n, be: (n, 0, 0)),
            ],
            out_specs=pl.BlockSpec((BM, h), lambda n, be: (n, 0)),
        ),
        out_shape=jax.ShapeDtypeStruct((NBM, h), jnp.float32),
        compiler_params=pltpu.CompilerParams(
            dimension_semantics=("arbitrary",)),
    )(block_expert, xg, wg, wu, down_proj, bg, bu, bd, rww)

    nt = t // NW
    out = pl.kernel(
        _sc_combine_body,
        out_type=jax.ShapeDtypeStruct((t, h), jnp.float32),
        mesh=mesh,
        scratch_types=[pltpu.VMEM((nt,), jnp.int32),
                       pltpu.VMEM((nt,), jnp.int32),
                       pltpu.VMEM((nt, h), jnp.float32),
                       pltpu.VMEM((nt, h), jnp.float32),
                       pltpu.SemaphoreType.DMA],
    )(yg, p0, p1)

    return out.reshape(b, s, h)


# speedup vs baseline: 1.4009x; 1.4009x over previous
import functools
import jax
import jax.numpy as jnp
from jax import lax
from jax.experimental import pallas as pl
from jax.experimental.pallas import tpu as pltpu
from jax.experimental.pallas import tpu_sc as plsc

NE = 8
ALPHA = 1.702
LIMIT = 7.0
BM = 128
NB = 2048 * 2 // BM + NE
NBM = NB * BM
NW = 32
NCHUNK = 4


def _router_body(x_ref, wT_ref, b_ref, idx_ref, wts_ref):
    x = x_ref[...]
    logits = jnp.dot(x, wT_ref[...], preferred_element_type=jnp.float32)
    logits = logits + b_ref[...]
    t = logits.shape[0]
    iota = lax.broadcasted_iota(jnp.int32, (t, NE), 1)
    m1 = jnp.max(logits, axis=1, keepdims=True)
    i1 = jnp.min(jnp.where(logits == m1, iota, NE), axis=1, keepdims=True)
    masked = jnp.where(iota == i1, -jnp.inf, logits)
    m2 = jnp.max(masked, axis=1, keepdims=True)
    i2 = jnp.min(jnp.where(masked == m2, iota, NE), axis=1, keepdims=True)
    r = jnp.exp(m2 - m1)
    w1 = 1.0 / (1.0 + r)
    idx_ref[...] = jnp.concatenate([i1, i2], axis=1)
    wts_ref[...] = jnp.concatenate([w1, 1.0 - w1], axis=1)


def _gmm_body(be_ref, xg_ref, wgu_ref, wd_ref, bg_ref, bu_ref, bd_ref,
              rww_ref, o_ref):
    x = xg_ref[...].astype(jnp.bfloat16)
    gate = jnp.dot(x, wgu_ref[0, 0], preferred_element_type=jnp.float32)
    up = jnp.dot(x, wgu_ref[0, 1], preferred_element_type=jnp.float32)
    gate = jnp.minimum(gate + bg_ref[0], LIMIT)
    up = jnp.clip(up + bu_ref[0], -LIMIT, LIMIT)
    glu = gate * (1.0 / (1.0 + jnp.exp(-ALPHA * gate)))
    act = ((up + 1.0) * glu).astype(jnp.bfloat16)
    y = jnp.dot(act, wd_ref[0], preferred_element_type=jnp.float32) + bd_ref[0]
    o_ref[...] = y * rww_ref[0]


def _sc_gather_body(x_hbm, tok_hbm, out_hbm, idx_v, b0, b1,
                    s0, s1, s2, s3):
    wid = lax.axis_index("s") * 2 + lax.axis_index("c")
    rows = NBM // NW
    cr = rows // NCHUNK
    base = wid * rows
    pltpu.sync_copy(tok_hbm.at[pl.ds(base, rows)], idx_v)
    bufs = (b0, b1)
    gsems = (s0, s1)
    wsems = (s2, s3)
    gcp = [None] * NCHUNK
    wcp = [None] * NCHUNK
    gcp[0] = pltpu.async_copy(x_hbm.at[idx_v.at[pl.ds(0, cr)]], b0, s0)
    gcp[1] = pltpu.async_copy(x_hbm.at[idx_v.at[pl.ds(cr, cr)]], b1, s1)
    for c in range(NCHUNK):
        gcp[c].wait()
        wcp[c] = pltpu.async_copy(bufs[c % 2],
                                  out_hbm.at[pl.ds(base + c * cr, cr)],
                                  wsems[c % 2])
        nxt = c + 2
        if nxt < NCHUNK:
            wcp[c].wait()
            gcp[nxt] = pltpu.async_copy(
                x_hbm.at[idx_v.at[pl.ds(nxt * cr, cr)]],
                bufs[nxt % 2], gsems[nxt % 2])
    for c in range(NCHUNK - 2, NCHUNK):
        wcp[c].wait()


def _sc_combine_body(yg_hbm, p0_hbm, p1_hbm, out_hbm, i0_v, i1_v, r0_v, r1_v,
                     sem):
    wid = lax.axis_index("s") * 2 + lax.axis_index("c")
    nt = 2048 // NW
    h = 768
    base = wid * nt
    pltpu.sync_copy(p0_hbm.at[pl.ds(base, nt)], i0_v)
    pltpu.sync_copy(p1_hbm.at[pl.ds(base, nt)], i1_v)
    pltpu.async_copy(yg_hbm.at[i0_v], r0_v, sem).wait()
    pltpu.async_copy(yg_hbm.at[i1_v], r1_v, sem).wait()

    def row(i, _):
        def chunk(j, _):
            r0_v[i, pl.ds(j * 16, 16)] = (r0_v[i, pl.ds(j * 16, 16)]
                                          + r1_v[i, pl.ds(j * 16, 16)])
            return _
        return lax.fori_loop(0, h // 16, chunk, _)

    lax.fori_loop(0, nt, row, None)
    pltpu.sync_copy(r0_v, out_hbm.at[pl.ds(base, nt)])


def kernel(hidden_states, router_weight, router_bias, gate_up_proj,
           gate_up_proj_bias, down_proj, down_proj_bias):
    b, s, h = hidden_states.shape
    t = b * s
    i_dim = down_proj.shape[1]
    x2 = hidden_states.reshape(t, h)

    top_idx, top_w = pl.pallas_call(
        _router_body,
        out_shape=(jax.ShapeDtypeStruct((t, 2), jnp.int32),
                   jax.ShapeDtypeStruct((t, 2), jnp.float32)),
        name="rt_router",
    )(x2, router_weight.T, router_bias.reshape(1, NE))

    e_flat = top_idx.reshape(-1)
    w_flat = top_w.reshape(-1)
    tok = lax.broadcasted_iota(jnp.int32, (t, 2), 0).reshape(-1)
    perm = jnp.argsort(e_flat, stable=True)
    e_sorted = e_flat[perm]
    sizes = jnp.sum((e_flat[:, None] == jnp.arange(NE)[None, :]), axis=0)
    starts = jnp.concatenate([jnp.zeros((1,), jnp.int32),
                              jnp.cumsum(sizes)[:-1].astype(jnp.int32)])
    nb_e = (sizes + BM - 1) // BM
    bounds = jnp.cumsum(nb_e)
    blk_start = bounds - nb_e
    block_expert = jnp.minimum(
        jnp.sum(jnp.arange(NB)[:, None] >= bounds[None, :], axis=1),
        NE - 1).astype(jnp.int32)
    rank = jnp.arange(2 * t, dtype=jnp.int32) - starts[e_sorted]
    pos = (blk_start[e_sorted] * BM).astype(jnp.int32) + rank
    row_token = jnp.zeros((NBM,), jnp.int32).at[pos].set(tok[perm])
    row_weight = jnp.zeros((NBM,), jnp.float32).at[pos].set(w_flat[perm])
    posj = jnp.zeros((2 * t,), jnp.int32).at[perm].set(pos).reshape(t, 2)
    p0 = posj[:, 0]
    p1 = posj[:, 1]

    mesh = plsc.VectorSubcoreMesh(core_axis_name="c", subcore_axis_name="s")
    cr = NBM // NW // NCHUNK
    xg = pl.kernel(
        _sc_gather_body,
        out_type=jax.ShapeDtypeStruct((NBM, h), jnp.float32),
        mesh=mesh,
        name="sc_gather_rows",
        scratch_types=[pltpu.VMEM((NBM // NW,), jnp.int32),
                       pltpu.VMEM((cr, h), jnp.float32),
                       pltpu.VMEM((cr, h), jnp.float32),
                       pltpu.SemaphoreType.DMA,
                       pltpu.SemaphoreType.DMA,
                       pltpu.SemaphoreType.DMA,
                       pltpu.SemaphoreType.DMA],
    )(x2, row_token)

    wgu = jnp.transpose(gate_up_proj.reshape(NE, h, i_dim, 2),
                        (0, 3, 1, 2)).astype(jnp.bfloat16)
    wd = down_proj.astype(jnp.bfloat16)
    bgu = gate_up_proj_bias.reshape(NE, 1, i_dim, 2)
    bg = bgu[..., 0]
    bu = bgu[..., 1]
    bd = down_proj_bias.reshape(NE, 1, h)
    rww = row_weight.reshape(NB, BM, 1)

    yg = pl.pallas_call(
        _gmm_body,
        grid_spec=pltpu.PrefetchScalarGridSpec(
            num_scalar_prefetch=1,
            grid=(NB,),
            in_specs=[
                pl.BlockSpec((BM, h), lambda n, be: (n, 0)),
                pl.BlockSpec((1, 2, h, i_dim), lambda n, be: (be[n], 0, 0, 0)),
                pl.BlockSpec((1, i_dim, h), lambda n, be: (be[n], 0, 0)),
                pl.BlockSpec((1, 1, i_dim), lambda n, be: (be[n], 0, 0)),
                pl.BlockSpec((1, 1, i_dim), lambda n, be: (be[n], 0, 0)),
                pl.BlockSpec((1, 1, h), lambda n, be: (be[n], 0, 0)),
                pl.BlockSpec((1, BM, 1), lambda n, be: (n, 0, 0)),
            ],
            out_specs=pl.BlockSpec((BM, h), lambda n, be: (n, 0)),
        ),
        out_shape=jax.ShapeDtypeStruct((NBM, h), jnp.float32),
        name="tc_grouped_ffn",
        compiler_params=pltpu.CompilerParams(
            dimension_semantics=("arbitrary",)),
    )(block_expert, xg, wgu, wd, bg, bu, bd, rww)

    nt = t // NW
    out = pl.kernel(
        _sc_combine_body,
        out_type=jax.ShapeDtypeStruct((t, h), jnp.float32),
        mesh=mesh,
        name="sc_combine_rows",
        scratch_types=[pltpu.VMEM((nt,), jnp.int32),
                       pltpu.VMEM((nt,), jnp.int32),
                       pltpu.VMEM((nt, h), jnp.float32),
                       pltpu.VMEM((nt, h), jnp.float32),
                       pltpu.SemaphoreType.DMA],
    )(yg, p0, p1)

    return out.reshape(b, s, h)
